# baseline (speedup 1.0000x reference)
import functools

import jax
import jax.numpy as jnp
from jax import lax
from jax.experimental import pallas as pl
from jax.experimental.pallas import tpu as pltpu
from jax.experimental.pallas import tpu_sc as plsc

B = 16384
NLAB = 4
NCLS = 10
QD = NCLS * NCLS

NC, NS = 2, 16
NW = NC * NS
BPW = B // NW
CH = 256
NCHUNK = BPW // CH

GAMMA = 0.25
ALPHA = GAMMA * (NLAB * 2) ** 2
BETA = ALPHA * 100.0 / 5.0


def _sc_gather_body(inst, pred,
                    l0, l1, l2, l3, qi0, qi1, qi2, qi3, qp0, qp1, qp2, qp3,
                    g0, g1, g2, g3, ri0, ri1, ri2, ri3, rp0, rp1, rp2, rp3,
                    idx_i, idx_p, log_buf, qi_buf, qp_buf,
                    sem_lg, sem_qi, sem_qp, sem_out):
  wid = lax.axis_index("s") * NC + lax.axis_index("c")
  base = wid * BPW
  pltpu.sync_copy(inst.at[pl.ds(base, BPW)], idx_i)
  pltpu.sync_copy(pred.at[pl.ds(base, BPW)], idx_p)
  ltabs = [l0, l1, l2, l3]
  qitabs = [qi0, qi1, qi2, qi3]
  qptabs = [qp0, qp1, qp2, qp3]
  gouts = [g0, g1, g2, g3]
  riouts = [ri0, ri1, ri2, ri3]
  rpouts = [rp0, rp1, rp2, rp3]
  for c in range(NCHUNK):
    rows = base + c * CH
    for l in range(NLAB):
      lt, qit, qpt = ltabs[l], qitabs[l], qptabs[l]

      def fire(i, _, lt=lt, qit=qit, qpt=qpt, c=c):
        vi = idx_i[pl.ds(c * CH + i * 16, 16)]
        vp = idx_p[pl.ds(c * CH + i * 16, 16)]
        for k in range(16):
          r = vi[k]
          rp = vp[k]
          j = i * 16 + k
          pltpu.async_copy(lt.at[r], log_buf.at[j], sem_lg)
          pltpu.async_copy(qit.at[r], qi_buf.at[j], sem_qi)
          pltpu.async_copy(qpt.at[rp], qp_buf.at[j], sem_qp)
        return 0

      lax.fori_loop(0, CH // 16, fire, 0)
      pltpu.make_async_copy(lt.at[pl.ds(0, CH)], log_buf, sem_lg).wait()
      pltpu.make_async_copy(qit.at[pl.ds(0, CH)], qi_buf, sem_qi).wait()
      pltpu.make_async_copy(qpt.at[pl.ds(0, CH)], qp_buf, sem_qp).wait()
      a = pltpu.async_copy(log_buf, gouts[l].at[pl.ds(rows, CH)], sem_out)
      b = pltpu.async_copy(qi_buf, riouts[l].at[pl.ds(rows, CH)], sem_out)
      d = pltpu.async_copy(qp_buf, rpouts[l].at[pl.ds(rows, CH)], sem_out)
      a.wait()
      b.wait()
      d.wait()


def _sc_gather(inst, pred, ltabs, qitabs, qptabs):
  mesh = plsc.VectorSubcoreMesh(core_axis_name="c", subcore_axis_name="s",
                                num_cores=NC, num_subcores=NS)
  out_type = ([jax.ShapeDtypeStruct((B, NCLS), jnp.float32)] * NLAB
              + [jax.ShapeDtypeStruct((B, QD), jnp.float32)] * (2 * NLAB))
  f = pl.kernel(
      _sc_gather_body,
      out_type=out_type,
      mesh=mesh,
      scratch_types=[
          pltpu.VMEM((BPW,), jnp.int32),
          pltpu.VMEM((BPW,), jnp.int32),
          pltpu.VMEM((CH, NCLS), jnp.float32),
          pltpu.VMEM((CH, QD), jnp.float32),
          pltpu.VMEM((CH, QD), jnp.float32),
          pltpu.SemaphoreType.DMA,
          pltpu.SemaphoreType.DMA,
          pltpu.SemaphoreType.DMA,
          pltpu.SemaphoreType.DMA,
      ],
  )
  return f(inst, pred, *ltabs, *qitabs, *qptabs)


_LN2 = 0.6931471805599453
_SQRT2 = 1.4142135623730951


def _fast_log(x):
  bits = jax.lax.bitcast_convert_type(x, jnp.int32)
  e = jax.lax.shift_right_logical(bits, 23) - 127
  mbits = jax.lax.bitwise_or(jax.lax.bitwise_and(bits, 0x7FFFFF), 0x3F800000)
  m = jax.lax.bitcast_convert_type(mbits, jnp.float32)
  big = m >= _SQRT2
  m = jnp.where(big, m * 0.5, m)
  ef = e.astype(jnp.float32) + big.astype(jnp.float32)
  t = m - 1.0
  z = t / (2.0 + t)
  z2 = z * z
  s = 2.0 * z * (1.0 + z2 * (1.0 / 3.0 + z2 * (0.2 + z2 * (1.0 / 7.0))))
  return ef * _LN2 + s


def _tc_math_body(*refs):
  gs = refs[0:4]
  qis = refs[4:8]
  qps = refs[8:12]
  seg40 = refs[12]
  seg100 = refs[13]
  ps = refs[14:18]
  cs = refs[18:22]
  regp = refs[22]

  x = jnp.concatenate([g[...] for g in gs], axis=1)
  m = jnp.max(x, axis=1, keepdims=True)
  e = jnp.exp(x - m)
  s = jax.lax.dot(e, seg40[...], precision=jax.lax.Precision.HIGHEST)
  out = x - m - _fast_log(s)
  for l in range(NLAB):
    ps[l][...] = out[:, NCLS * l:NCLS * (l + 1)]

  blk = qis[0].shape[0]
  for l in range(NLAB):
    a = qis[l][...]
    bq = qps[l][...]
    y = a + bq
    m2 = jnp.max(y, axis=1, keepdims=True)
    e2 = jnp.exp(y - m2)
    s2 = jax.lax.dot(e2, seg100[...], precision=jax.lax.Precision.HIGHEST)
    cf = y - m2 - _fast_log(s2)
    cs[l][...] = cf.reshape(blk, NCLS, NCLS)
    regp[0, 0, 2 * l] = jnp.sum(a * a) * (BETA * 0.5)
    regp[0, 0, 2 * l + 1] = jnp.sum(bq * bq) * (ALPHA * 0.5)


def _tc_math(gs, qirows, qprows):
  ngrid = 64
  blk = B // ngrid
  i40 = lax.broadcasted_iota(jnp.int32, (NCLS * NLAB, NCLS * NLAB), 0)
  j40 = lax.broadcasted_iota(jnp.int32, (NCLS * NLAB, NCLS * NLAB), 1)
  seg40 = (i40 // NCLS == j40 // NCLS).astype(jnp.float32)
  i100 = lax.broadcasted_iota(jnp.int32, (QD, QD), 0)
  j100 = lax.broadcasted_iota(jnp.int32, (QD, QD), 1)
  seg100 = (i100 // NCLS == j100 // NCLS).astype(jnp.float32)

  in_specs = (
      [pl.BlockSpec((blk, NCLS), lambda i: (i, 0))] * NLAB
      + [pl.BlockSpec((blk, QD), lambda i: (i, 0))] * (2 * NLAB)
      + [pl.BlockSpec((NCLS * NLAB, NCLS * NLAB), lambda i: (0, 0))]
      + [pl.BlockSpec((QD, QD), lambda i: (0, 0))]
  )
  out_specs = (
      [pl.BlockSpec((blk, NCLS), lambda i: (i, 0))] * NLAB
      + [pl.BlockSpec((blk, NCLS, NCLS), lambda i: (i, 0, 0))] * NLAB
      + [pl.BlockSpec((1, 1, 8), lambda i: (i, 0, 0),
                      memory_space=pltpu.SMEM)]
  )
  out_shape = (
      [jax.ShapeDtypeStruct((B, NCLS), jnp.float32)] * NLAB
      + [jax.ShapeDtypeStruct((B, NCLS, NCLS), jnp.float32)] * NLAB
      + [jax.ShapeDtypeStruct((ngrid, 1, 8), jnp.float32)]
  )
  return pl.pallas_call(
      _tc_math_body,
      grid=(ngrid,),
      in_specs=in_specs,
      out_specs=out_specs,
      out_shape=out_shape,
  )(*gs, *qirows, *qprows, seg40, seg100)


def kernel(instances, predictors, labels, inst_emb,
           logits_0, logits_1, logits_2, logits_3,
           qi_0, qi_1, qi_2, qi_3, qp_0, qp_1, qp_2, qp_3):
  del labels, inst_emb
  inst = instances.astype(jnp.int32)
  pred = predictors.astype(jnp.int32)
  ltabs = [logits_0, logits_1, logits_2, logits_3]
  qitabs = [qi_0, qi_1, qi_2, qi_3]
  qptabs = [qp_0, qp_1, qp_2, qp_3]

  _DIAG_XLA_GATHER = True
  if _DIAG_XLA_GATHER:
    gs = [jnp.take(t, inst, axis=0) for t in ltabs]
    qirows = [jnp.take(t, inst, axis=0) for t in qitabs]
    qprows = [jnp.take(t, pred, axis=0) for t in qptabs]
  else:
    gathered = _sc_gather(inst, pred, ltabs, qitabs, qptabs)
    gs = gathered[0:4]
    qirows = gathered[4:8]
    qprows = gathered[8:12]

  outs = _tc_math(gs, qirows, qprows)
  preds = tuple(outs[0:4])
  confs = tuple(outs[4:8])
  reg_terms = jnp.sum(outs[8], axis=(0, 1))
  return (preds, confs, reg_terms)


# speedup vs baseline: 2.4814x; 2.4814x over previous
import functools

import jax
import jax.numpy as jnp
from jax import lax
from jax.experimental import pallas as pl
from jax.experimental.pallas import tpu as pltpu
from jax.experimental.pallas import tpu_sc as plsc

B = 16384
NLAB = 4
NCLS = 10
QD = NCLS * NCLS

NC, NS = 2, 16
NW = NC * NS
BPW = B // NW
CH = 256
NCHUNK = BPW // CH

GAMMA = 0.25
ALPHA = GAMMA * (NLAB * 2) ** 2
BETA = ALPHA * 100.0 / 5.0


def _sc_gather_body(inst, pred,
                    l0, l1, l2, l3, qi0, qi1, qi2, qi3, qp0, qp1, qp2, qp3,
                    g0, g1, g2, g3, ri0, ri1, ri2, ri3, rp0, rp1, rp2, rp3,
                    idx_i, idx_p, log_buf, qi_buf, qp_buf,
                    sem_lg, sem_qi, sem_qp, sem_out):
  wid = lax.axis_index("s") * NC + lax.axis_index("c")
  base = wid * BPW
  pltpu.sync_copy(inst.at[pl.ds(base, BPW)], idx_i)
  pltpu.sync_copy(pred.at[pl.ds(base, BPW)], idx_p)
  ltabs = [l0, l1, l2, l3]
  qitabs = [qi0, qi1, qi2, qi3]
  qptabs = [qp0, qp1, qp2, qp3]
  gouts = [g0, g1, g2, g3]
  riouts = [ri0, ri1, ri2, ri3]
  rpouts = [rp0, rp1, rp2, rp3]
  for c in range(NCHUNK):
    rows = base + c * CH
    for l in range(NLAB):
      lt, qit, qpt = ltabs[l], qitabs[l], qptabs[l]

      def fire(i, _, lt=lt, qit=qit, qpt=qpt, c=c):
        vi = idx_i[pl.ds(c * CH + i * 16, 16)]
        vp = idx_p[pl.ds(c * CH + i * 16, 16)]
        for k in range(16):
          r = vi[k]
          rp = vp[k]
          j = i * 16 + k
          pltpu.async_copy(lt.at[r], log_buf.at[j], sem_lg)
          pltpu.async_copy(qit.at[r], qi_buf.at[j], sem_qi)
          pltpu.async_copy(qpt.at[rp], qp_buf.at[j], sem_qp)
        return 0

      lax.fori_loop(0, CH // 16, fire, 0)
      pltpu.make_async_copy(lt.at[pl.ds(0, CH)], log_buf, sem_lg).wait()
      pltpu.make_async_copy(qit.at[pl.ds(0, CH)], qi_buf, sem_qi).wait()
      pltpu.make_async_copy(qpt.at[pl.ds(0, CH)], qp_buf, sem_qp).wait()
      a = pltpu.async_copy(log_buf, gouts[l].at[pl.ds(rows, CH)], sem_out)
      b = pltpu.async_copy(qi_buf, riouts[l].at[pl.ds(rows, CH)], sem_out)
      d = pltpu.async_copy(qp_buf, rpouts[l].at[pl.ds(rows, CH)], sem_out)
      a.wait()
      b.wait()
      d.wait()


def _sc_gather(inst, pred, ltabs, qitabs, qptabs):
  mesh = plsc.VectorSubcoreMesh(core_axis_name="c", subcore_axis_name="s",
                                num_cores=NC, num_subcores=NS)
  out_type = ([jax.ShapeDtypeStruct((B, NCLS), jnp.float32)] * NLAB
              + [jax.ShapeDtypeStruct((B, QD), jnp.float32)] * (2 * NLAB))
  f = pl.kernel(
      _sc_gather_body,
      out_type=out_type,
      mesh=mesh,
      scratch_types=[
          pltpu.VMEM((BPW,), jnp.int32),
          pltpu.VMEM((BPW,), jnp.int32),
          pltpu.VMEM((CH, NCLS), jnp.float32),
          pltpu.VMEM((CH, QD), jnp.float32),
          pltpu.VMEM((CH, QD), jnp.float32),
          pltpu.SemaphoreType.DMA,
          pltpu.SemaphoreType.DMA,
          pltpu.SemaphoreType.DMA,
          pltpu.SemaphoreType.DMA,
      ],
  )
  return f(inst, pred, *ltabs, *qitabs, *qptabs)


_LN2 = 0.6931471805599453
_SQRT2 = 1.4142135623730951


def _fast_log(x):
  bits = jax.lax.bitcast_convert_type(x, jnp.int32)
  e = jax.lax.shift_right_logical(bits, 23) - 127
  mbits = jax.lax.bitwise_or(jax.lax.bitwise_and(bits, 0x7FFFFF), 0x3F800000)
  m = jax.lax.bitcast_convert_type(mbits, jnp.float32)
  big = m >= _SQRT2
  m = jnp.where(big, m * 0.5, m)
  ef = e.astype(jnp.float32) + big.astype(jnp.float32)
  t = m - 1.0
  z = t / (2.0 + t)
  z2 = z * z
  s = 2.0 * z * (1.0 + z2 * (1.0 / 3.0 + z2 * (0.2 + z2 * (1.0 / 7.0))))
  return ef * _LN2 + s


def _tc_math_body(*refs):
  gs = refs[0:4]
  qis = refs[4:8]
  qps = refs[8:12]
  seg40 = refs[12]
  seg100 = refs[13]
  ps = refs[14:18]
  cs = refs[18:22]
  regp = refs[22]

  x = jnp.concatenate([g[...] for g in gs], axis=1)
  m = jnp.max(x, axis=1, keepdims=True)
  e = jnp.exp(x - m)
  s = jax.lax.dot(e, seg40[...], precision=jax.lax.Precision.HIGHEST)
  out = x - m - _fast_log(s)
  for l in range(NLAB):
    ps[l][...] = out[:, NCLS * l:NCLS * (l + 1)]

  blk = qis[0].shape[0]
  for l in range(NLAB):
    a = qis[l][...]
    bq = qps[l][...]
    y = a + bq
    m2 = jnp.max(y, axis=1, keepdims=True)
    e2 = jnp.exp(y - m2)
    s2 = jax.lax.dot(e2, seg100[...], precision=jax.lax.Precision.HIGHEST)
    cf = y - m2 - _fast_log(s2)
    cs[l][...] = cf
    regp[0, 0, 2 * l] = jnp.sum(a * a) * (BETA * 0.5)
    regp[0, 0, 2 * l + 1] = jnp.sum(bq * bq) * (ALPHA * 0.5)


def _tc_math(gs, qirows, qprows):
  ngrid = 64
  blk = B // ngrid
  i40 = lax.broadcasted_iota(jnp.int32, (NCLS * NLAB, NCLS * NLAB), 0)
  j40 = lax.broadcasted_iota(jnp.int32, (NCLS * NLAB, NCLS * NLAB), 1)
  seg40 = (i40 // NCLS == j40 // NCLS).astype(jnp.float32)
  i100 = lax.broadcasted_iota(jnp.int32, (QD, QD), 0)
  j100 = lax.broadcasted_iota(jnp.int32, (QD, QD), 1)
  seg100 = (i100 // NCLS == j100 // NCLS).astype(jnp.float32)

  in_specs = (
      [pl.BlockSpec((blk, NCLS), lambda i: (i, 0))] * NLAB
      + [pl.BlockSpec((blk, QD), lambda i: (i, 0))] * (2 * NLAB)
      + [pl.BlockSpec((NCLS * NLAB, NCLS * NLAB), lambda i: (0, 0))]
      + [pl.BlockSpec((QD, QD), lambda i: (0, 0))]
  )
  out_specs = (
      [pl.BlockSpec((blk, NCLS), lambda i: (i, 0))] * NLAB
      + [pl.BlockSpec((blk, QD), lambda i: (i, 0))] * NLAB
      + [pl.BlockSpec((1, 1, 8), lambda i: (i, 0, 0),
                      memory_space=pltpu.SMEM)]
  )
  out_shape = (
      [jax.ShapeDtypeStruct((B, NCLS), jnp.float32)] * NLAB
      + [jax.ShapeDtypeStruct((B, QD), jnp.float32)] * NLAB
      + [jax.ShapeDtypeStruct((ngrid, 1, 8), jnp.float32)]
  )
  return pl.pallas_call(
      _tc_math_body,
      grid=(ngrid,),
      in_specs=in_specs,
      out_specs=out_specs,
      out_shape=out_shape,
  )(*gs, *qirows, *qprows, seg40, seg100)


def kernel(instances, predictors, labels, inst_emb,
           logits_0, logits_1, logits_2, logits_3,
           qi_0, qi_1, qi_2, qi_3, qp_0, qp_1, qp_2, qp_3):
  del labels, inst_emb
  inst = instances.astype(jnp.int32)
  pred = predictors.astype(jnp.int32)
  ltabs = [logits_0, logits_1, logits_2, logits_3]
  qitabs = [qi_0, qi_1, qi_2, qi_3]
  qptabs = [qp_0, qp_1, qp_2, qp_3]

  gathered = _sc_gather(inst, pred, ltabs, qitabs, qptabs)
  gs = gathered[0:4]
  qirows = gathered[4:8]
  qprows = gathered[8:12]

  outs = _tc_math(gs, qirows, qprows)
  preds = tuple(outs[0:4])
  confs = tuple(x.reshape(B, NCLS, NCLS) for x in outs[4:8])
  reg_terms = jnp.sum(outs[8], axis=(0, 1))
  return (preds, confs, reg_terms)


# speedup vs baseline: 2.7470x; 1.1071x over previous
import functools

import jax
import jax.numpy as jnp
from jax import lax
from jax.experimental import pallas as pl
from jax.experimental.pallas import tpu as pltpu
from jax.experimental.pallas import tpu_sc as plsc

B = 16384
NLAB = 4
NCLS = 10
QD = NCLS * NCLS

NC, NS = 2, 16
NW = NC * NS
BPW = B // NW
CH = 256
NCHUNK = BPW // CH

GAMMA = 0.25
ALPHA = GAMMA * (NLAB * 2) ** 2
BETA = ALPHA * 100.0 / 5.0


def _sc_gather_body(inst, pred,
                    l0, l1, l2, l3, qi0, qi1, qi2, qi3, qp0, qp1, qp2, qp3,
                    g0, g1, g2, g3, ri0, ri1, ri2, ri3, rp0, rp1, rp2, rp3,
                    idx_i, idx_p, log_buf, qi_buf, qp_buf,
                    sem_lg, sem_qi, sem_qp, sem_out):
  wid = lax.axis_index("s") * NC + lax.axis_index("c")
  base = wid * BPW
  pltpu.sync_copy(inst.at[pl.ds(base, BPW)], idx_i)
  pltpu.sync_copy(pred.at[pl.ds(base, BPW)], idx_p)
  ltabs = [l0, l1, l2, l3]
  qitabs = [qi0, qi1, qi2, qi3]
  qptabs = [qp0, qp1, qp2, qp3]
  gouts = [g0, g1, g2, g3]
  riouts = [ri0, ri1, ri2, ri3]
  rpouts = [rp0, rp1, rp2, rp3]
  for c in range(NCHUNK):
    rows = base + c * CH
    for l in range(NLAB):
      lt, qit, qpt = ltabs[l], qitabs[l], qptabs[l]

      def fire(i, _, lt=lt, qit=qit, qpt=qpt, c=c):
        vi = idx_i[pl.ds(c * CH + i * 16, 16)]
        vp = idx_p[pl.ds(c * CH + i * 16, 16)]
        for k in range(16):
          r = vi[k]
          rp = vp[k]
          j = i * 16 + k
          pltpu.async_copy(lt.at[r], log_buf.at[j], sem_lg)
          pltpu.async_copy(qit.at[r], qi_buf.at[j], sem_qi)
          pltpu.async_copy(qpt.at[rp], qp_buf.at[j], sem_qp)
        return 0

      lax.fori_loop(0, CH // 16, fire, 0)
      pltpu.make_async_copy(lt.at[pl.ds(0, CH)], log_buf, sem_lg).wait()
      pltpu.make_async_copy(qit.at[pl.ds(0, CH)], qi_buf, sem_qi).wait()
      pltpu.make_async_copy(qpt.at[pl.ds(0, CH)], qp_buf, sem_qp).wait()
      a = pltpu.async_copy(log_buf, gouts[l].at[pl.ds(rows, CH)], sem_out)
      b = pltpu.async_copy(qi_buf, riouts[l].at[pl.ds(rows, CH)], sem_out)
      d = pltpu.async_copy(qp_buf, rpouts[l].at[pl.ds(rows, CH)], sem_out)
      a.wait()
      b.wait()
      d.wait()


def _sc_gather(inst, pred, ltabs, qitabs, qptabs):
  mesh = plsc.VectorSubcoreMesh(core_axis_name="c", subcore_axis_name="s",
                                num_cores=NC, num_subcores=NS)
  out_type = ([jax.ShapeDtypeStruct((B, NCLS), jnp.float32)] * NLAB
              + [jax.ShapeDtypeStruct((B, QD), jnp.float32)] * (2 * NLAB))
  f = pl.kernel(
      _sc_gather_body,
      out_type=out_type,
      mesh=mesh,
      scratch_types=[
          pltpu.VMEM((BPW,), jnp.int32),
          pltpu.VMEM((BPW,), jnp.int32),
          pltpu.VMEM((CH, NCLS), jnp.float32),
          pltpu.VMEM((CH, QD), jnp.float32),
          pltpu.VMEM((CH, QD), jnp.float32),
          pltpu.SemaphoreType.DMA,
          pltpu.SemaphoreType.DMA,
          pltpu.SemaphoreType.DMA,
          pltpu.SemaphoreType.DMA,
      ],
  )
  return f(inst, pred, *ltabs, *qitabs, *qptabs)


_LN2 = 0.6931471805599453
_SQRT2 = 1.4142135623730951


def _fast_log(x):
  bits = jax.lax.bitcast_convert_type(x, jnp.int32)
  e = jax.lax.shift_right_logical(bits, 23) - 127
  mbits = jax.lax.bitwise_or(jax.lax.bitwise_and(bits, 0x7FFFFF), 0x3F800000)
  m = jax.lax.bitcast_convert_type(mbits, jnp.float32)
  big = m >= _SQRT2
  m = jnp.where(big, m * 0.5, m)
  ef = e.astype(jnp.float32) + big.astype(jnp.float32)
  t = m - 1.0
  z = t / (2.0 + t)
  z2 = z * z
  s = 2.0 * z * (1.0 + z2 * (1.0 / 3.0 + z2 * (0.2 + z2 * (1.0 / 7.0))))
  return ef * _LN2 + s


def _tc_math_body(*refs):
  gs = refs[0:4]
  qis = refs[4:8]
  qps = refs[8:12]
  seg40 = refs[12]
  seg100 = refs[13]
  ps = refs[14:18]
  cs = refs[18:22]
  regp = refs[22]

  x = jnp.concatenate([g[...] for g in gs], axis=1)
  m = jnp.max(x, axis=1, keepdims=True)
  e = jnp.exp(x - m)
  s = jax.lax.dot(e, seg40[...], precision=jax.lax.Precision.HIGHEST)
  out = x - m - _fast_log(s)
  for l in range(NLAB):
    ps[l][...] = out[:, NCLS * l:NCLS * (l + 1)]

  blk = qis[0].shape[0]
  for l in range(NLAB):
    a = qis[l][...]
    bq = qps[l][...]
    y = a + bq
    m2 = jnp.max(y, axis=1, keepdims=True)
    e2 = jnp.exp(y - m2)
    s2 = jax.lax.dot(e2, seg100[...], precision=jax.lax.Precision.HIGHEST)
    cf = y - m2 - _fast_log(s2)
    cft = cf.T
    for k in range(NCLS):
      cs[l][k] = cft[NCLS * k:NCLS * (k + 1), :]
    regp[0, 0, 2 * l] = jnp.sum(a * a) * (BETA * 0.5)
    regp[0, 0, 2 * l + 1] = jnp.sum(bq * bq) * (ALPHA * 0.5)


def _tc_math(gs, qirows, qprows):
  ngrid = 64
  blk = B // ngrid
  i40 = lax.broadcasted_iota(jnp.int32, (NCLS * NLAB, NCLS * NLAB), 0)
  j40 = lax.broadcasted_iota(jnp.int32, (NCLS * NLAB, NCLS * NLAB), 1)
  seg40 = (i40 // NCLS == j40 // NCLS).astype(jnp.float32)
  i100 = lax.broadcasted_iota(jnp.int32, (QD, QD), 0)
  j100 = lax.broadcasted_iota(jnp.int32, (QD, QD), 1)
  seg100 = (i100 // NCLS == j100 // NCLS).astype(jnp.float32)

  in_specs = (
      [pl.BlockSpec((blk, NCLS), lambda i: (i, 0))] * NLAB
      + [pl.BlockSpec((blk, QD), lambda i: (i, 0))] * (2 * NLAB)
      + [pl.BlockSpec((NCLS * NLAB, NCLS * NLAB), lambda i: (0, 0))]
      + [pl.BlockSpec((QD, QD), lambda i: (0, 0))]
  )
  out_specs = (
      [pl.BlockSpec((blk, NCLS), lambda i: (i, 0))] * NLAB
      + [pl.BlockSpec((NCLS, NCLS, blk), lambda i: (0, 0, i))] * NLAB
      + [pl.BlockSpec((1, 1, 8), lambda i: (i, 0, 0),
                      memory_space=pltpu.SMEM)]
  )
  out_shape = (
      [jax.ShapeDtypeStruct((B, NCLS), jnp.float32)] * NLAB
      + [jax.ShapeDtypeStruct((NCLS, NCLS, B), jnp.float32)] * NLAB
      + [jax.ShapeDtypeStruct((ngrid, 1, 8), jnp.float32)]
  )
  return pl.pallas_call(
      _tc_math_body,
      grid=(ngrid,),
      in_specs=in_specs,
      out_specs=out_specs,
      out_shape=out_shape,
  )(*gs, *qirows, *qprows, seg40, seg100)


def kernel(instances, predictors, labels, inst_emb,
           logits_0, logits_1, logits_2, logits_3,
           qi_0, qi_1, qi_2, qi_3, qp_0, qp_1, qp_2, qp_3):
  del labels, inst_emb
  inst = instances.astype(jnp.int32)
  pred = predictors.astype(jnp.int32)
  ltabs = [logits_0, logits_1, logits_2, logits_3]
  qitabs = [qi_0, qi_1, qi_2, qi_3]
  qptabs = [qp_0, qp_1, qp_2, qp_3]

  gathered = _sc_gather(inst, pred, ltabs, qitabs, qptabs)
  gs = gathered[0:4]
  qirows = gathered[4:8]
  qprows = gathered[8:12]

  outs = _tc_math(gs, qirows, qprows)
  preds = tuple(outs[0:4])
  confs = tuple(jnp.transpose(x, (2, 0, 1)) for x in outs[4:8])
  reg_terms = jnp.sum(outs[8], axis=(0, 1))
  return (preds, confs, reg_terms)


# speedup vs baseline: 2.7498x; 1.0010x over previous
import functools

import jax
import jax.numpy as jnp
from jax import lax
from jax.experimental import pallas as pl
from jax.experimental.pallas import tpu as pltpu
from jax.experimental.pallas import tpu_sc as plsc

B = 16384
NLAB = 4
NCLS = 10
QD = NCLS * NCLS

NC, NS = 2, 16
NW = NC * NS
BPW = B // NW
CH = 256
NCHUNK = BPW // CH

GAMMA = 0.25
ALPHA = GAMMA * (NLAB * 2) ** 2
BETA = ALPHA * 100.0 / 5.0


def _sc_gather_body(inst, pred,
                    l0, l1, l2, l3, qi0, qi1, qi2, qi3, qp0, qp1, qp2, qp3,
                    g0, g1, g2, g3, ri0, ri1, ri2, ri3, rp0, rp1, rp2, rp3,
                    idx_i, idx_p, log_buf, qi_buf, qp_buf,
                    sem_lg, sem_qi, sem_qp, sem_out):
  wid = lax.axis_index("s") * NC + lax.axis_index("c")
  base = wid * BPW
  pltpu.sync_copy(inst.at[pl.ds(base, BPW)], idx_i)
  pltpu.sync_copy(pred.at[pl.ds(base, BPW)], idx_p)
  ltabs = [l0, l1, l2, l3]
  qitabs = [qi0, qi1, qi2, qi3]
  qptabs = [qp0, qp1, qp2, qp3]
  gouts = [g0, g1, g2, g3]
  riouts = [ri0, ri1, ri2, ri3]
  rpouts = [rp0, rp1, rp2, rp3]
  for c in range(NCHUNK):
    rows = base + c * CH
    for l in range(NLAB):
      lt, qit, qpt = ltabs[l], qitabs[l], qptabs[l]

      def fire(i, _, lt=lt, qit=qit, qpt=qpt, c=c):
        vi = idx_i[pl.ds(c * CH + i * 16, 16)]
        vp = idx_p[pl.ds(c * CH + i * 16, 16)]
        for k in range(16):
          r = vi[k]
          rp = vp[k]
          j = i * 16 + k
          pltpu.async_copy(lt.at[r], log_buf.at[j], sem_lg)
          pltpu.async_copy(qit.at[r], qi_buf.at[j], sem_qi)
          pltpu.async_copy(qpt.at[rp], qp_buf.at[j], sem_qp)
        return 0

      lax.fori_loop(0, CH // 16, fire, 0)
      pltpu.make_async_copy(lt.at[pl.ds(0, CH)], log_buf, sem_lg).wait()
      pltpu.make_async_copy(qit.at[pl.ds(0, CH)], qi_buf, sem_qi).wait()
      pltpu.make_async_copy(qpt.at[pl.ds(0, CH)], qp_buf, sem_qp).wait()
      a = pltpu.async_copy(log_buf, gouts[l].at[pl.ds(rows, CH)], sem_out)
      b = pltpu.async_copy(qi_buf, riouts[l].at[pl.ds(rows, CH)], sem_out)
      d = pltpu.async_copy(qp_buf, rpouts[l].at[pl.ds(rows, CH)], sem_out)
      a.wait()
      b.wait()
      d.wait()


def _sc_gather(inst, pred, ltabs, qitabs, qptabs):
  mesh = plsc.VectorSubcoreMesh(core_axis_name="c", subcore_axis_name="s",
                                num_cores=NC, num_subcores=NS)
  out_type = ([jax.ShapeDtypeStruct((B, NCLS), jnp.float32)] * NLAB
              + [jax.ShapeDtypeStruct((B, QD), jnp.float32)] * (2 * NLAB))
  f = pl.kernel(
      _sc_gather_body,
      out_type=out_type,
      mesh=mesh,
      scratch_types=[
          pltpu.VMEM((BPW,), jnp.int32),
          pltpu.VMEM((BPW,), jnp.int32),
          pltpu.VMEM((CH, NCLS), jnp.float32),
          pltpu.VMEM((CH, QD), jnp.float32),
          pltpu.VMEM((CH, QD), jnp.float32),
          pltpu.SemaphoreType.DMA,
          pltpu.SemaphoreType.DMA,
          pltpu.SemaphoreType.DMA,
          pltpu.SemaphoreType.DMA,
      ],
  )
  return f(inst, pred, *ltabs, *qitabs, *qptabs)


_LN2 = 0.6931471805599453
_SQRT2 = 1.4142135623730951


def _fast_log(x):
  bits = jax.lax.bitcast_convert_type(x, jnp.int32)
  e = jax.lax.shift_right_logical(bits, 23) - 127
  mbits = jax.lax.bitwise_or(jax.lax.bitwise_and(bits, 0x7FFFFF), 0x3F800000)
  m = jax.lax.bitcast_convert_type(mbits, jnp.float32)
  big = m >= _SQRT2
  m = jnp.where(big, m * 0.5, m)
  ef = e.astype(jnp.float32) + big.astype(jnp.float32)
  t = m - 1.0
  z = t / (2.0 + t)
  z2 = z * z
  s = 2.0 * z * (1.0 + z2 * (1.0 / 3.0 + z2 * (0.2 + z2 * (1.0 / 7.0))))
  return ef * _LN2 + s


def _tc_math_body(*refs):
  gs = refs[0:4]
  qis = refs[4:8]
  qps = refs[8:12]
  seg40 = refs[12]
  seg100 = refs[13]
  ps = refs[14:18]
  cs = refs[18:22]
  regp = refs[22]

  x = jnp.concatenate([g[...] for g in gs], axis=1)
  m = jnp.max(x, axis=1, keepdims=True)
  e = jnp.exp(x - m)
  s = jax.lax.dot(e, seg40[...], precision=jax.lax.Precision.HIGHEST)
  out = x - m - _fast_log(s)
  for l in range(NLAB):
    ps[l][...] = out[:, NCLS * l:NCLS * (l + 1)]

  blk = qis[0].shape[0]
  for l in range(NLAB):
    a = qis[l][...]
    bq = qps[l][...]
    y = a + bq
    m2 = jnp.max(y, axis=1, keepdims=True)
    e2 = jnp.exp(y - m2)
    s2 = jax.lax.dot(e2, seg100[...], precision=jax.lax.Precision.HIGHEST)
    cf = y - m2 - _fast_log(s2)
    cft = cf.T
    for k in range(NCLS):
      cs[l][k] = cft[NCLS * k:NCLS * (k + 1), :]
    regp[0, 0, 2 * l] = jnp.sum(a * a) * (BETA * 0.5)
    regp[0, 0, 2 * l + 1] = jnp.sum(bq * bq) * (ALPHA * 0.5)


def _tc_math(gs, qirows, qprows):
  ngrid = 64
  blk = B // ngrid
  i40 = lax.broadcasted_iota(jnp.int32, (NCLS * NLAB, NCLS * NLAB), 0)
  j40 = lax.broadcasted_iota(jnp.int32, (NCLS * NLAB, NCLS * NLAB), 1)
  seg40 = (i40 // NCLS == j40 // NCLS).astype(jnp.float32)
  i100 = lax.broadcasted_iota(jnp.int32, (QD, QD), 0)
  j100 = lax.broadcasted_iota(jnp.int32, (QD, QD), 1)
  seg100 = (i100 // NCLS == j100 // NCLS).astype(jnp.float32)

  in_specs = (
      [pl.BlockSpec((blk, NCLS), lambda i: (i, 0))] * NLAB
      + [pl.BlockSpec((blk, QD), lambda i: (i, 0))] * (2 * NLAB)
      + [pl.BlockSpec((NCLS * NLAB, NCLS * NLAB), lambda i: (0, 0))]
      + [pl.BlockSpec((QD, QD), lambda i: (0, 0))]
  )
  out_specs = (
      [pl.BlockSpec((blk, NCLS), lambda i: (i, 0))] * NLAB
      + [pl.BlockSpec((NCLS, NCLS, blk), lambda i: (0, 0, i))] * NLAB
      + [pl.BlockSpec((1, 1, 8), lambda i: (i, 0, 0),
                      memory_space=pltpu.SMEM)]
  )
  out_shape = (
      [jax.ShapeDtypeStruct((B, NCLS), jnp.float32)] * NLAB
      + [jax.ShapeDtypeStruct((NCLS, NCLS, B), jnp.float32)] * NLAB
      + [jax.ShapeDtypeStruct((ngrid, 1, 8), jnp.float32)]
  )
  return pl.pallas_call(
      _tc_math_body,
      grid=(ngrid,),
      in_specs=in_specs,
      out_specs=out_specs,
      out_shape=out_shape,
  )(*gs, *qirows, *qprows, seg40, seg100)


def kernel(instances, predictors, labels, inst_emb,
           logits_0, logits_1, logits_2, logits_3,
           qi_0, qi_1, qi_2, qi_3, qp_0, qp_1, qp_2, qp_3):
  del labels, inst_emb
  inst = instances.astype(jnp.int32)
  pred = predictors.astype(jnp.int32)
  ltabs = [logits_0, logits_1, logits_2, logits_3]
  qitabs = [qi_0, qi_1, qi_2, qi_3]
  qptabs = [qp_0, qp_1, qp_2, qp_3]

  gathered = _sc_gather(inst, pred, ltabs, qitabs, qptabs)
  gs = gathered[0:4]
  qirows = gathered[4:8]
  qprows = gathered[8:12]

  outs = _tc_math(gs, qirows, qprows)
  preds = tuple(outs[0:4])
  confs = tuple(
      jax.lax.reshape(x, (B, NCLS, NCLS), dimensions=(2, 0, 1))
      for x in outs[4:8])
  reg_terms = jnp.sum(outs[8], axis=(0, 1))
  return (preds, confs, reg_terms)


# speedup vs baseline: 2.8687x; 1.0433x over previous
import functools

import jax
import jax.numpy as jnp
from jax import lax
from jax.experimental import pallas as pl
from jax.experimental.pallas import tpu as pltpu
from jax.experimental.pallas import tpu_sc as plsc

B = 16384
NLAB = 4
NCLS = 10
QD = NCLS * NCLS

NC, NS = 2, 16
NW = NC * NS
BPW = B // NW
CH = 256
NCHUNK = BPW // CH

GAMMA = 0.25
ALPHA = GAMMA * (NLAB * 2) ** 2
BETA = ALPHA * 100.0 / 5.0


def _sc_gather_body(inst, pred, lt, qit, qpt,
                    gout, riout, rpout,
                    idx_i, idx_p, log_buf, qi_buf, qp_buf,
                    sem_lg, sem_qi, sem_qp, sem_out):
  wid = lax.axis_index("s") * NC + lax.axis_index("c")
  base = wid * BPW
  pltpu.sync_copy(inst.at[pl.ds(base, BPW)], idx_i)
  pltpu.sync_copy(pred.at[pl.ds(base, BPW)], idx_p)
  for c in range(NCHUNK):
    rows = base + c * CH

    def fire(i, _, c=c):
      vi = idx_i[pl.ds(c * CH + i * 16, 16)]
      vp = idx_p[pl.ds(c * CH + i * 16, 16)]
      for k in range(16):
        r = vi[k]
        rp = vp[k]
        j = i * 16 + k
        pltpu.async_copy(lt.at[r], log_buf.at[j], sem_lg)
        pltpu.async_copy(qit.at[r], qi_buf.at[j], sem_qi)
        pltpu.async_copy(qpt.at[rp], qp_buf.at[j], sem_qp)
      return 0

    lax.fori_loop(0, CH // 16, fire, 0)
    pltpu.make_async_copy(lt.at[pl.ds(0, CH)], log_buf, sem_lg).wait()
    pltpu.make_async_copy(qit.at[pl.ds(0, CH)], qi_buf, sem_qi).wait()
    pltpu.make_async_copy(qpt.at[pl.ds(0, CH)], qp_buf, sem_qp).wait()
    a = pltpu.async_copy(log_buf, gout.at[pl.ds(rows, CH)], sem_out)
    b = pltpu.async_copy(qi_buf, riout.at[pl.ds(rows, CH)], sem_out)
    d = pltpu.async_copy(qp_buf, rpout.at[pl.ds(rows, CH)], sem_out)
    a.wait()
    b.wait()
    d.wait()


def _sc_gather_one(inst, pred, lt, qit, qpt):
  mesh = plsc.VectorSubcoreMesh(core_axis_name="c", subcore_axis_name="s",
                                num_cores=NC, num_subcores=NS)
  out_type = [jax.ShapeDtypeStruct((B, NCLS), jnp.float32),
              jax.ShapeDtypeStruct((B, QD), jnp.float32),
              jax.ShapeDtypeStruct((B, QD), jnp.float32)]
  f = pl.kernel(
      _sc_gather_body,
      out_type=out_type,
      mesh=mesh,
      scratch_types=[
          pltpu.VMEM((BPW,), jnp.int32),
          pltpu.VMEM((BPW,), jnp.int32),
          pltpu.VMEM((CH, NCLS), jnp.float32),
          pltpu.VMEM((CH, QD), jnp.float32),
          pltpu.VMEM((CH, QD), jnp.float32),
          pltpu.SemaphoreType.DMA,
          pltpu.SemaphoreType.DMA,
          pltpu.SemaphoreType.DMA,
          pltpu.SemaphoreType.DMA,
      ],
  )
  return f(inst, pred, lt, qit, qpt)


def _sc_gather(inst, pred, ltabs, qitabs, qptabs):
  gs, qirows, qprows = [], [], []
  for l in range(NLAB):
    g, ri, rp = _sc_gather_one(inst, pred, ltabs[l], qitabs[l], qptabs[l])
    gs.append(g)
    qirows.append(ri)
    qprows.append(rp)
  return gs + qirows + qprows


_LN2 = 0.6931471805599453
_SQRT2 = 1.4142135623730951


def _fast_log(x):
  bits = jax.lax.bitcast_convert_type(x, jnp.int32)
  e = jax.lax.shift_right_logical(bits, 23) - 127
  mbits = jax.lax.bitwise_or(jax.lax.bitwise_and(bits, 0x7FFFFF), 0x3F800000)
  m = jax.lax.bitcast_convert_type(mbits, jnp.float32)
  big = m >= _SQRT2
  m = jnp.where(big, m * 0.5, m)
  ef = e.astype(jnp.float32) + big.astype(jnp.float32)
  t = m - 1.0
  z = t / (2.0 + t)
  z2 = z * z
  s = 2.0 * z * (1.0 + z2 * (1.0 / 3.0 + z2 * (0.2 + z2 * (1.0 / 7.0))))
  return ef * _LN2 + s


def _tc_math_body(*refs):
  gs = refs[0:4]
  qis = refs[4:8]
  qps = refs[8:12]
  seg40 = refs[12]
  seg100 = refs[13]
  ps = refs[14:18]
  cs = refs[18:22]
  regp = refs[22]

  x = jnp.concatenate([g[...] for g in gs], axis=1)
  m = jnp.max(x, axis=1, keepdims=True)
  e = jnp.exp(x - m)
  s = jax.lax.dot(e, seg40[...], precision=jax.lax.Precision.HIGHEST)
  out = x - m - _fast_log(s)
  for l in range(NLAB):
    ps[l][...] = out[:, NCLS * l:NCLS * (l + 1)]

  blk = qis[0].shape[0]
  for l in range(NLAB):
    a = qis[l][...]
    bq = qps[l][...]
    y = a + bq
    m2 = jnp.max(y, axis=1, keepdims=True)
    e2 = jnp.exp(y - m2)
    s2 = jax.lax.dot(e2, seg100[...], precision=jax.lax.Precision.HIGHEST)
    cf = y - m2 - _fast_log(s2)
    cft = cf.T
    for k in range(NCLS):
      cs[l][k] = cft[NCLS * k:NCLS * (k + 1), :]
    regp[0, 0, 2 * l] = jnp.sum(a * a) * (BETA * 0.5)
    regp[0, 0, 2 * l + 1] = jnp.sum(bq * bq) * (ALPHA * 0.5)


def _tc_math(gs, qirows, qprows):
  ngrid = 64
  blk = B // ngrid
  i40 = lax.broadcasted_iota(jnp.int32, (NCLS * NLAB, NCLS * NLAB), 0)
  j40 = lax.broadcasted_iota(jnp.int32, (NCLS * NLAB, NCLS * NLAB), 1)
  seg40 = (i40 // NCLS == j40 // NCLS).astype(jnp.float32)
  i100 = lax.broadcasted_iota(jnp.int32, (QD, QD), 0)
  j100 = lax.broadcasted_iota(jnp.int32, (QD, QD), 1)
  seg100 = (i100 // NCLS == j100 // NCLS).astype(jnp.float32)

  in_specs = (
      [pl.BlockSpec((blk, NCLS), lambda i: (i, 0))] * NLAB
      + [pl.BlockSpec((blk, QD), lambda i: (i, 0))] * (2 * NLAB)
      + [pl.BlockSpec((NCLS * NLAB, NCLS * NLAB), lambda i: (0, 0))]
      + [pl.BlockSpec((QD, QD), lambda i: (0, 0))]
  )
  out_specs = (
      [pl.BlockSpec((blk, NCLS), lambda i: (i, 0))] * NLAB
      + [pl.BlockSpec((NCLS, NCLS, blk), lambda i: (0, 0, i))] * NLAB
      + [pl.BlockSpec((1, 1, 8), lambda i: (i, 0, 0),
                      memory_space=pltpu.SMEM)]
  )
  out_shape = (
      [jax.ShapeDtypeStruct((B, NCLS), jnp.float32)] * NLAB
      + [jax.ShapeDtypeStruct((NCLS, NCLS, B), jnp.float32)] * NLAB
      + [jax.ShapeDtypeStruct((ngrid, 1, 8), jnp.float32)]
  )
  return pl.pallas_call(
      _tc_math_body,
      grid=(ngrid,),
      in_specs=in_specs,
      out_specs=out_specs,
      out_shape=out_shape,
  )(*gs, *qirows, *qprows, seg40, seg100)


def kernel(instances, predictors, labels, inst_emb,
           logits_0, logits_1, logits_2, logits_3,
           qi_0, qi_1, qi_2, qi_3, qp_0, qp_1, qp_2, qp_3):
  del labels, inst_emb
  inst = instances.astype(jnp.int32)
  pred = predictors.astype(jnp.int32)
  ltabs = [logits_0, logits_1, logits_2, logits_3]
  qitabs = [qi_0, qi_1, qi_2, qi_3]
  qptabs = [qp_0, qp_1, qp_2, qp_3]

  gathered = _sc_gather(inst, pred, ltabs, qitabs, qptabs)
  gs = gathered[0:4]
  qirows = gathered[4:8]
  qprows = gathered[8:12]

  outs = _tc_math(gs, qirows, qprows)
  preds = tuple(outs[0:4])
  confs = tuple(
      jax.lax.reshape(x, (B, NCLS, NCLS), dimensions=(2, 0, 1))
      for x in outs[4:8])
  reg_terms = jnp.sum(outs[8], axis=(0, 1))
  return (preds, confs, reg_terms)


# speedup vs baseline: 2.8954x; 1.0093x over previous
import functools

import jax
import jax.numpy as jnp
from jax import lax
from jax.experimental import pallas as pl
from jax.experimental.pallas import tpu as pltpu
from jax.experimental.pallas import tpu_sc as plsc

B = 16384
NLAB = 4
NCLS = 10
QD = NCLS * NCLS

NC, NS = 2, 16
NW = NC * NS
BPW = B // NW
CH = 256
NCHUNK = BPW // CH

GAMMA = 0.25
ALPHA = GAMMA * (NLAB * 2) ** 2
BETA = ALPHA * 100.0 / 5.0


def _sc_gather_body(inst, pred, lt, qit, qpt,
                    gout, riout, rpout,
                    idx_i, idx_p, log_buf, qi_buf, qp_buf,
                    sem_lg, sem_qi, sem_qp, sem_out):
  wid = lax.axis_index("s") * NC + lax.axis_index("c")
  base = wid * BPW
  pltpu.sync_copy(inst.at[pl.ds(base, BPW)], idx_i)
  pltpu.sync_copy(pred.at[pl.ds(base, BPW)], idx_p)
  for c in range(NCHUNK):
    rows = base + c * CH

    def fire(i, _, c=c):
      vi = idx_i[pl.ds(c * CH + i * 16, 16)]
      vp = idx_p[pl.ds(c * CH + i * 16, 16)]
      for k in range(16):
        r = vi[k]
        rp = vp[k]
        j = i * 16 + k
        pltpu.async_copy(lt.at[r], log_buf.at[j], sem_lg)
        pltpu.async_copy(qit.at[r], qi_buf.at[j], sem_qi)
        pltpu.async_copy(qpt.at[rp], qp_buf.at[j], sem_qp)
      return 0

    lax.fori_loop(0, CH // 16, fire, 0)
    pltpu.make_async_copy(lt.at[pl.ds(0, CH)], log_buf, sem_lg).wait()
    pltpu.make_async_copy(qit.at[pl.ds(0, CH)], qi_buf, sem_qi).wait()
    pltpu.make_async_copy(qpt.at[pl.ds(0, CH)], qp_buf, sem_qp).wait()
    a = pltpu.async_copy(log_buf, gout.at[pl.ds(rows, CH)], sem_out)
    b = pltpu.async_copy(qi_buf, riout.at[pl.ds(rows, CH)], sem_out)
    d = pltpu.async_copy(qp_buf, rpout.at[pl.ds(rows, CH)], sem_out)
    a.wait()
    b.wait()
    d.wait()


def _sc_gather_one(inst, pred, lt, qit, qpt):
  mesh = plsc.VectorSubcoreMesh(core_axis_name="c", subcore_axis_name="s",
                                num_cores=NC, num_subcores=NS)
  out_type = [jax.ShapeDtypeStruct((B, NCLS), jnp.float32),
              jax.ShapeDtypeStruct((B, QD), jnp.float32),
              jax.ShapeDtypeStruct((B, QD), jnp.float32)]
  f = pl.kernel(
      _sc_gather_body,
      out_type=out_type,
      mesh=mesh,
      scratch_types=[
          pltpu.VMEM((BPW,), jnp.int32),
          pltpu.VMEM((BPW,), jnp.int32),
          pltpu.VMEM((CH, NCLS), jnp.float32),
          pltpu.VMEM((CH, QD), jnp.float32),
          pltpu.VMEM((CH, QD), jnp.float32),
          pltpu.SemaphoreType.DMA,
          pltpu.SemaphoreType.DMA,
          pltpu.SemaphoreType.DMA,
          pltpu.SemaphoreType.DMA,
      ],
  )
  return f(inst, pred, lt, qit, qpt)


def _sc_gather(inst, pred, ltabs, qitabs, qptabs):
  gs, qirows, qprows = [], [], []
  for l in range(NLAB):
    g, ri, rp = _sc_gather_one(inst, pred, ltabs[l], qitabs[l], qptabs[l])
    gs.append(g)
    qirows.append(ri)
    qprows.append(rp)
  return gs + qirows + qprows


_LN2 = 0.6931471805599453
_SQRT2 = 1.4142135623730951


def _fast_log(x):
  bits = jax.lax.bitcast_convert_type(x, jnp.int32)
  e = jax.lax.shift_right_logical(bits, 23) - 127
  mbits = jax.lax.bitwise_or(jax.lax.bitwise_and(bits, 0x7FFFFF), 0x3F800000)
  m = jax.lax.bitcast_convert_type(mbits, jnp.float32)
  big = m >= _SQRT2
  m = jnp.where(big, m * 0.5, m)
  ef = e.astype(jnp.float32) + big.astype(jnp.float32)
  t = m - 1.0
  z = t / (2.0 + t)
  z2 = z * z
  s = 2.0 * z * (1.0 + z2 * (1.0 / 3.0 + z2 * (0.2 + z2 * (1.0 / 7.0))))
  return ef * _LN2 + s


def _tc_math_body(*refs):
  gs = refs[0:4]
  qis = refs[4:8]
  qps = refs[8:12]
  seg40 = refs[12]
  seg100 = refs[13]
  ps = refs[14:18]
  cs = refs[18:22]
  regp = refs[22]

  x = jnp.concatenate([g[...] for g in gs], axis=1)
  m = jnp.max(x, axis=1, keepdims=True)
  e = jnp.exp(x - m)
  s = jax.lax.dot(e, seg40[...], precision=jax.lax.Precision.HIGHEST)
  out = x - m - _fast_log(s)
  for l in range(NLAB):
    ps[l][...] = out[:, NCLS * l:NCLS * (l + 1)]

  blk = qis[0].shape[0]
  for l in range(NLAB):
    a = qis[l][...]
    bq = qps[l][...]
    y = a + bq
    m2 = jnp.max(y, axis=1, keepdims=True)
    e2 = jnp.exp(y - m2)
    s2 = jax.lax.dot(e2, seg100[...], precision=jax.lax.Precision.HIGHEST)
    cf = y - m2 - _fast_log(s2)
    cft = cf.T
    for k in range(NCLS):
      cs[l][k] = cft[NCLS * k:NCLS * (k + 1), :]
    regp[0, 0, 2 * l] = jnp.sum(a * a) * (BETA * 0.5)
    regp[0, 0, 2 * l + 1] = jnp.sum(bq * bq) * (ALPHA * 0.5)


def _tc_math(gs, qirows, qprows):
  ngrid = 32
  blk = B // ngrid
  i40 = lax.broadcasted_iota(jnp.int32, (NCLS * NLAB, NCLS * NLAB), 0)
  j40 = lax.broadcasted_iota(jnp.int32, (NCLS * NLAB, NCLS * NLAB), 1)
  seg40 = (i40 // NCLS == j40 // NCLS).astype(jnp.float32)
  i100 = lax.broadcasted_iota(jnp.int32, (QD, QD), 0)
  j100 = lax.broadcasted_iota(jnp.int32, (QD, QD), 1)
  seg100 = (i100 // NCLS == j100 // NCLS).astype(jnp.float32)

  in_specs = (
      [pl.BlockSpec((blk, NCLS), lambda i: (i, 0))] * NLAB
      + [pl.BlockSpec((blk, QD), lambda i: (i, 0))] * (2 * NLAB)
      + [pl.BlockSpec((NCLS * NLAB, NCLS * NLAB), lambda i: (0, 0))]
      + [pl.BlockSpec((QD, QD), lambda i: (0, 0))]
  )
  out_specs = (
      [pl.BlockSpec((blk, NCLS), lambda i: (i, 0))] * NLAB
      + [pl.BlockSpec((NCLS, NCLS, blk), lambda i: (0, 0, i))] * NLAB
      + [pl.BlockSpec((1, 1, 8), lambda i: (i, 0, 0),
                      memory_space=pltpu.SMEM)]
  )
  out_shape = (
      [jax.ShapeDtypeStruct((B, NCLS), jnp.float32)] * NLAB
      + [jax.ShapeDtypeStruct((NCLS, NCLS, B), jnp.float32)] * NLAB
      + [jax.ShapeDtypeStruct((ngrid, 1, 8), jnp.float32)]
  )
  return pl.pallas_call(
      _tc_math_body,
      grid=(ngrid,),
      in_specs=in_specs,
      out_specs=out_specs,
      out_shape=out_shape,
  )(*gs, *qirows, *qprows, seg40, seg100)


def kernel(instances, predictors, labels, inst_emb,
           logits_0, logits_1, logits_2, logits_3,
           qi_0, qi_1, qi_2, qi_3, qp_0, qp_1, qp_2, qp_3):
  del labels, inst_emb
  inst = instances.astype(jnp.int32)
  pred = predictors.astype(jnp.int32)
  ltabs = [logits_0, logits_1, logits_2, logits_3]
  qitabs = [qi_0, qi_1, qi_2, qi_3]
  qptabs = [qp_0, qp_1, qp_2, qp_3]

  gathered = _sc_gather(inst, pred, ltabs, qitabs, qptabs)
  gs = gathered[0:4]
  qirows = gathered[4:8]
  qprows = gathered[8:12]

  outs = _tc_math(gs, qirows, qprows)
  preds = tuple(outs[0:4])
  confs = tuple(
      jax.lax.reshape(x, (B, NCLS, NCLS), dimensions=(2, 0, 1))
      for x in outs[4:8])
  reg_terms = jnp.sum(outs[8], axis=(0, 1))
  return (preds, confs, reg_terms)
